# Initial kernel scaffold; baseline (speedup 1.0000x reference)
import functools

import jax
import jax.numpy as jnp
from jax import lax
from jax.experimental import pallas as pl
from jax.experimental.pallas import tpu as pltpu
from jax.experimental.pallas import tpu_sc as plsc

N = 10000
D = 128
E = 320000
NC = 2
NS = 16
NW = NC * NS
EPW = E // NW
CHUNK = 80
NCHUNK = EPW // CHUNK
ROWS_PER_TILE = N // NS


def _sc_agg_body(x_hbm, src_hbm, dst_hbm, zeros_hbm, out_hbm,
                 src_v, dst_v, rows_v, agg_sh, gsem):
    cid = lax.axis_index("c")
    sid = lax.axis_index("s")
    wid = sid * NC + cid

    r0 = sid * ROWS_PER_TILE
    pltpu.sync_copy(zeros_hbm.at[pl.ds(r0, ROWS_PER_TILE)],
                    agg_sh.at[pl.ds(r0, ROWS_PER_TILE)])
    plsc.subcore_barrier()

    base = wid * EPW

    def body(i, carry):
        off = base + i * CHUNK
        pltpu.sync_copy(src_hbm.at[pl.ds(off, CHUNK)], src_v)
        pltpu.sync_copy(dst_hbm.at[pl.ds(off, CHUNK)], dst_v)
        pltpu.async_copy(x_hbm.at[src_v], rows_v, gsem).wait()
        pltpu.sync_copy(rows_v, agg_sh.at[dst_v], add=True)
        return carry

    lax.fori_loop(0, NCHUNK, body, 0)
    plsc.subcore_barrier()
    pltpu.sync_copy(agg_sh.at[pl.ds(r0, ROWS_PER_TILE)],
                    out_hbm.at[cid, pl.ds(r0, ROWS_PER_TILE)])


def _sc_agg(x, src, dst, zeros):
    mesh = plsc.VectorSubcoreMesh(core_axis_name="c", subcore_axis_name="s")
    k = pl.kernel(
        _sc_agg_body,
        out_type=jax.ShapeDtypeStruct((NC, N, D), jnp.float32),
        mesh=mesh,
        scratch_types=[
            pltpu.VMEM((CHUNK,), jnp.int32),
            pltpu.VMEM((CHUNK,), jnp.int32),
            pltpu.VMEM((CHUNK, D), jnp.float32),
            pltpu.VMEM_SHARED((N, D), jnp.float32),
            pltpu.SemaphoreType.DMA,
        ],
    )
    return k(x, src, dst, zeros)


def _dense_body(relu, x_ref, p_ref, wr_ref, wn_ref, b_ref, o_ref):
    agg = p_ref[0] + p_ref[1]
    dn = (((1,), (1,)), ((), ()))
    acc = lax.dot_general(x_ref[...], wr_ref[...], dn,
                          preferred_element_type=jnp.float32)
    acc += lax.dot_general(agg, wn_ref[...], dn,
                           preferred_element_type=jnp.float32)
    acc += b_ref[...]
    if relu:
        acc = jnp.maximum(acc, 0.0)
    o_ref[...] = acc


def _dense_layer(x, parts, w_root, w_nbr, b, relu):
    R = 500
    grid = (N // R,)
    b2 = b.reshape(1, D)
    return pl.pallas_call(
        functools.partial(_dense_body, relu),
        grid=grid,
        in_specs=[
            pl.BlockSpec((R, D), lambda i: (i, 0)),
            pl.BlockSpec((NC, R, D), lambda i: (0, i, 0)),
            pl.BlockSpec((D, D), lambda i: (0, 0)),
            pl.BlockSpec((D, D), lambda i: (0, 0)),
            pl.BlockSpec((1, D), lambda i: (0, 0)),
        ],
        out_specs=pl.BlockSpec((R, D), lambda i: (i, 0)),
        out_shape=jax.ShapeDtypeStruct((N, D), jnp.float32),
    )(x, parts, w_root, w_nbr, b2)


def kernel(x, edge_index, W1_root, W1_nbr, b1, W2_root, W2_nbr, b2):
    src = edge_index[0]
    dst = edge_index[1]
    zeros = jnp.zeros((N, D), jnp.float32)

    p1 = _sc_agg(x, src, dst, zeros)
    h = _dense_layer(x, p1, W1_root, W1_nbr, b1, relu=True)
    p2 = _sc_agg(h, src, dst, zeros)
    out = _dense_layer(h, p2, W2_root, W2_nbr, b2, relu=False)
    return out


# speedup vs baseline: 9.3247x; 9.3247x over previous
import functools

import jax
import jax.numpy as jnp
from jax import lax
from jax.experimental import pallas as pl
from jax.experimental.pallas import tpu as pltpu
from jax.experimental.pallas import tpu_sc as plsc

N = 10000
D = 128
E = 320000
NC = 2
NS = 16
NW = NC * NS
EPW = E // NW
CHUNK = 80
NCHUNK = EPW // CHUNK
NP = 10112
ROWS_PER_TILE = NP // NS


def _sc_agg_body(x_hbm, src_hbm, dst_hbm, zeros_hbm, out_hbm,
                 src_v, dst_v, rows_a, rows_b, agg_sh, gsem_a, gsem_b):
    cid = lax.axis_index("c")
    sid = lax.axis_index("s")
    wid = sid * NC + cid

    pltpu.sync_copy(src_hbm.at[pl.ds(wid * EPW, EPW)], src_v)
    pltpu.sync_copy(dst_hbm.at[wid], dst_v)

    r0 = sid * ROWS_PER_TILE
    pltpu.sync_copy(zeros_hbm.at[pl.ds(r0, ROWS_PER_TILE)],
                    agg_sh.at[pl.ds(r0, ROWS_PER_TILE)])
    plsc.subcore_barrier()

    pltpu.async_copy(x_hbm.at[src_v.at[pl.ds(0, CHUNK)]], rows_a, gsem_a)

    def body(j, carry):
        i0 = 2 * j
        pltpu.make_async_copy(x_hbm.at[src_v.at[pl.ds(i0 * CHUNK, CHUNK)]], rows_a, gsem_a).wait()
        pltpu.async_copy(x_hbm.at[src_v.at[pl.ds((i0 + 1) * CHUNK, CHUNK)]], rows_b, gsem_b)
        pltpu.sync_copy(rows_a, agg_sh.at[dst_v.at[i0]], add=True)
        pltpu.make_async_copy(x_hbm.at[src_v.at[pl.ds((i0 + 1) * CHUNK, CHUNK)]], rows_b, gsem_b).wait()
        pltpu.async_copy(x_hbm.at[src_v.at[pl.ds((i0 + 2) * CHUNK, CHUNK)]], rows_a, gsem_a)
        pltpu.sync_copy(rows_b, agg_sh.at[dst_v.at[i0 + 1]], add=True)
        return carry

    lax.fori_loop(0, (NCHUNK - 1) // 2, body, 0)
    last = NCHUNK - 1
    pltpu.make_async_copy(x_hbm.at[src_v.at[pl.ds(last * CHUNK, CHUNK)]], rows_a, gsem_a).wait()
    pltpu.sync_copy(rows_a, agg_sh.at[dst_v.at[last]], add=True)

    plsc.subcore_barrier()
    pltpu.sync_copy(agg_sh.at[pl.ds(r0, ROWS_PER_TILE)],
                    out_hbm.at[cid, pl.ds(r0, ROWS_PER_TILE)])


def _sc_agg(x, src3, dst3, zeros):
    mesh = plsc.VectorSubcoreMesh(core_axis_name="c", subcore_axis_name="s")
    k = pl.kernel(
        _sc_agg_body,
        out_type=jax.ShapeDtypeStruct((NC, NP, D), jnp.float32),
        mesh=mesh,
        scratch_types=[
            pltpu.VMEM((EPW,), jnp.int32),
            pltpu.VMEM((NCHUNK, CHUNK), jnp.int32),
            pltpu.VMEM((CHUNK, D), jnp.float32),
            pltpu.VMEM((CHUNK, D), jnp.float32),
            pltpu.VMEM_SHARED((NP, D), jnp.float32),
            pltpu.SemaphoreType.DMA,
            pltpu.SemaphoreType.DMA,
        ],
    )
    return k(x, src3, dst3, zeros)


def _dense_body(relu, x_ref, p_ref, wr_ref, wn_ref, b_ref, o_ref):
    agg = p_ref[0] + p_ref[1]
    dn = (((1,), (1,)), ((), ()))
    acc = lax.dot_general(x_ref[...], wr_ref[...], dn,
                          preferred_element_type=jnp.float32)
    acc += lax.dot_general(agg, wn_ref[...], dn,
                           preferred_element_type=jnp.float32)
    acc += b_ref[...]
    if relu:
        acc = jnp.maximum(acc, 0.0)
    o_ref[...] = acc


def _dense_layer(x, parts, w_root, w_nbr, b, relu):
    R = 1000
    grid = (N // R,)
    b2 = b.reshape(1, D)
    return pl.pallas_call(
        functools.partial(_dense_body, relu),
        grid=grid,
        in_specs=[
            pl.BlockSpec((R, D), lambda i: (i, 0)),
            pl.BlockSpec((NC, R, D), lambda i: (0, i, 0)),
            pl.BlockSpec((D, D), lambda i: (0, 0)),
            pl.BlockSpec((D, D), lambda i: (0, 0)),
            pl.BlockSpec((1, D), lambda i: (0, 0)),
        ],
        out_specs=pl.BlockSpec((R, D), lambda i: (i, 0)),
        out_shape=jax.ShapeDtypeStruct((N, D), jnp.float32),
    )(x, parts, w_root, w_nbr, b2)


def kernel(x, edge_index, W1_root, W1_nbr, b1, W2_root, W2_nbr, b2):
    src_flat = edge_index[0]
    dst3 = edge_index[1].reshape(NW, NCHUNK, CHUNK)
    zeros = jnp.zeros((NP, D), jnp.float32)

    p1 = _sc_agg(x, src_flat, dst3, zeros)
    h = _dense_layer(x, p1, W1_root, W1_nbr, b1, relu=True)
    p2 = _sc_agg(h, src_flat, dst3, zeros)
    out = _dense_layer(h, p2, W2_root, W2_nbr, b2, relu=False)
    return out
